# Initial kernel scaffold; baseline (speedup 1.0000x reference)
import jax
import jax.numpy as jnp
from jax.experimental import pallas as pl


def kernel(x, nbr_idx, dw_w1, pw_w1, bn1_g, bn1_b, bn1_m, bn1_v, dw_w2, pw_w2, bn2_g, bn2_b, bn2_m, bn2_v):
    raise NotImplementedError("write your pallas kernel here")


# speedup vs baseline: 1.0570x; 1.0570x over previous
import functools

import jax
import jax.numpy as jnp
from jax.experimental import pallas as pl
from jax.experimental.pallas import tpu as pltpu

EPS = 1e-5
B = 400


def _half_kernel(idx_hbm, src_any, center_blk, res_blk, w8, wc, pw, sc, bi,
                 out, idx_smem, gbuf, sem_i, sem_g, *, nblk, relu_gather,
                 relu_out, add_residual):
    b = pl.program_id(0)

    @pl.when(b < nblk)
    def _compute():
        cp = pltpu.make_async_copy(idx_hbm.at[b], idx_smem, sem_i)
        cp.start()
        cp.wait()

        def issue(i, carry):
            for kk in range(8):
                j = kk * B + i
                t = idx_smem[j]
                pltpu.make_async_copy(
                    src_any.at[pl.ds(t, 1), :],
                    gbuf.at[pl.ds(j, 1), :],
                    sem_g,
                ).start()
            return carry

        jax.lax.fori_loop(0, B, issue, 0)
        pltpu.make_async_copy(gbuf, gbuf, sem_g).wait()

        ctr = center_blk[...]
        if relu_gather:
            ctr = jnp.maximum(ctr, 0.0)
        acc = ctr * wc[...]
        for kk in range(8):
            g = gbuf[kk * B:(kk + 1) * B, :]
            if relu_gather:
                g = jnp.maximum(g, 0.0)
            acc = acc + g * w8[kk:kk + 1, :]

        mm = jnp.dot(acc.astype(jnp.bfloat16), pw[...],
                     preferred_element_type=jnp.float32)
        h = mm * sc[...] + bi[...]
        if relu_out:
            h = jnp.maximum(h, 0.0)
        if add_residual:
            h = h + res_blk[...]
        out[...] = h

    if nblk < pl.num_programs(0):
        @pl.when(b >= nblk)
        def _zero_tail():
            out[...] = jnp.zeros(out.shape, out.dtype)


def _run_half(idx_flat, src_pad, residual, w8, wc, pw_bf16, sc, bi, *,
              n_out_rows, nblk, grid, relu_gather, relu_out, add_residual):
    kern = functools.partial(
        _half_kernel, nblk=nblk, relu_gather=relu_gather, relu_out=relu_out,
        add_residual=add_residual)
    blk = lambda b: (b, 0)
    zero = lambda b: (0, 0)
    if not add_residual:
        res_spec = pl.BlockSpec((1, 256), zero)
        residual = sc
    else:
        res_spec = pl.BlockSpec((B, 256), blk)
    return pl.pallas_call(
        kern,
        grid=(grid,),
        in_specs=[
            pl.BlockSpec(memory_space=pl.ANY),
            pl.BlockSpec(memory_space=pl.ANY),
            pl.BlockSpec((B, 256), blk),
            res_spec,
            pl.BlockSpec((8, 256), zero),
            pl.BlockSpec((1, 256), zero),
            pl.BlockSpec((256, 256), zero),
            pl.BlockSpec((1, 256), zero),
            pl.BlockSpec((1, 256), zero),
        ],
        out_specs=pl.BlockSpec((B, 256), blk),
        out_shape=jax.ShapeDtypeStruct((n_out_rows, 256), jnp.float32),
        scratch_shapes=[
            pltpu.SMEM((8 * B,), jnp.int32),
            pltpu.VMEM((8 * B, 256), jnp.float32),
            pltpu.SemaphoreType.DMA,
            pltpu.SemaphoreType.DMA,
        ],
        compiler_params=pltpu.CompilerParams(
            dimension_semantics=("parallel",),
        ),
    )(idx_flat, src_pad, src_pad, residual, w8, wc, pw_bf16, sc, bi)


def kernel(x, nbr_idx, dw_w1, pw_w1, bn1_g, bn1_b, bn1_m, bn1_v,
           dw_w2, pw_w2, bn2_g, bn2_b, bn2_m, bn2_v):
    n, c = x.shape
    assert c == 256 and n % B == 0
    nblk = n // B

    idxp = jnp.where(nbr_idx < 0, jnp.int32(n), nbr_idx.astype(jnp.int32))
    sel = jnp.concatenate([idxp[:4], idxp[5:]], axis=0)
    sel = jnp.pad(sel, ((0, 0), (0, B)))
    idx_flat = sel.reshape(8, nblk + 1, B).transpose(1, 0, 2)
    idx_flat = idx_flat.reshape(nblk + 1, 8 * B)

    xpad = jnp.concatenate([x, jnp.zeros((B, c), jnp.float32)], axis=0)

    s1 = (bn1_g * jax.lax.rsqrt(bn1_v + EPS)).reshape(1, c)
    o1 = (bn1_b - bn1_m * s1[0]).reshape(1, c)
    s2 = (bn2_g * jax.lax.rsqrt(bn2_v + EPS)).reshape(1, c)
    o2 = (bn2_b - bn2_m * s2[0]).reshape(1, c)

    w8_1 = jnp.concatenate([dw_w1[:4], dw_w1[5:]], axis=0)
    wc_1 = dw_w1[4:5]
    w8_2 = jnp.concatenate([dw_w2[:4], dw_w2[5:]], axis=0)
    wc_2 = dw_w2[4:5]

    pw1b = pw_w1.astype(jnp.bfloat16)
    pw2b = pw_w2.astype(jnp.bfloat16)

    h1pad = _run_half(
        idx_flat, xpad, xpad, w8_1, wc_1, pw1b, s1, o1,
        n_out_rows=n + B, nblk=nblk, grid=nblk + 1,
        relu_gather=True, relu_out=True, add_residual=False)

    out = _run_half(
        idx_flat, h1pad, x, w8_2, wc_2, pw2b, s2, o2,
        n_out_rows=n, nblk=nblk, grid=nblk,
        relu_gather=False, relu_out=False, add_residual=True)
    return out


# speedup vs baseline: 1.0574x; 1.0005x over previous
import functools

import jax
import jax.numpy as jnp
from jax.experimental import pallas as pl
from jax.experimental.pallas import tpu as pltpu

EPS = 1e-5
B = 400


def _half_kernel(idx_hbm, src_any, center_blk, res_blk, w8, wc, pw, sc, bi,
                 out, idx_smem, gbuf, sem_i, sem_g, *, nblk, relu_gather,
                 relu_out, add_residual):
    b = pl.program_id(0)

    @pl.when(b < nblk)
    def _compute():
        cp = pltpu.make_async_copy(idx_hbm.at[b], idx_smem, sem_i)
        cp.start()
        cp.wait()

        def issue(i, carry):
            for kk in range(8):
                j = kk * B + i
                t = idx_smem[j]
                pltpu.make_async_copy(
                    src_any.at[pl.ds(t, 1), :],
                    gbuf.at[pl.ds(j, 1), :],
                    sem_g,
                ).start(priority=kk % 2)
            return carry

        jax.lax.fori_loop(0, B, issue, 0)
        pltpu.make_async_copy(gbuf, gbuf, sem_g).wait()

        ctr = center_blk[...]
        if relu_gather:
            ctr = jnp.maximum(ctr, 0.0)
        acc = ctr * wc[...]
        for kk in range(8):
            g = gbuf[kk * B:(kk + 1) * B, :]
            if relu_gather:
                g = jnp.maximum(g, 0.0)
            acc = acc + g * w8[kk:kk + 1, :]

        mm = jnp.dot(acc.astype(jnp.bfloat16), pw[...],
                     preferred_element_type=jnp.float32)
        h = mm * sc[...] + bi[...]
        if relu_out:
            h = jnp.maximum(h, 0.0)
        if add_residual:
            h = h + res_blk[...]
        out[...] = h

    if nblk < pl.num_programs(0):
        @pl.when(b >= nblk)
        def _zero_tail():
            out[...] = jnp.zeros(out.shape, out.dtype)


def _run_half(idx_flat, src_pad, residual, w8, wc, pw_bf16, sc, bi, *,
              n_out_rows, nblk, grid, relu_gather, relu_out, add_residual):
    kern = functools.partial(
        _half_kernel, nblk=nblk, relu_gather=relu_gather, relu_out=relu_out,
        add_residual=add_residual)
    blk = lambda b: (b, 0)
    zero = lambda b: (0, 0)
    if not add_residual:
        res_spec = pl.BlockSpec((1, 256), zero)
        residual = sc
    else:
        res_spec = pl.BlockSpec((B, 256), blk)
    return pl.pallas_call(
        kern,
        grid=(grid,),
        in_specs=[
            pl.BlockSpec(memory_space=pl.ANY),
            pl.BlockSpec(memory_space=pl.ANY),
            pl.BlockSpec((B, 256), blk),
            res_spec,
            pl.BlockSpec((8, 256), zero),
            pl.BlockSpec((1, 256), zero),
            pl.BlockSpec((256, 256), zero),
            pl.BlockSpec((1, 256), zero),
            pl.BlockSpec((1, 256), zero),
        ],
        out_specs=pl.BlockSpec((B, 256), blk),
        out_shape=jax.ShapeDtypeStruct((n_out_rows, 256), jnp.float32),
        scratch_shapes=[
            pltpu.SMEM((8 * B,), jnp.int32),
            pltpu.VMEM((8 * B, 256), jnp.float32),
            pltpu.SemaphoreType.DMA,
            pltpu.SemaphoreType.DMA,
        ],
        compiler_params=pltpu.CompilerParams(
            dimension_semantics=("parallel",),
        ),
    )(idx_flat, src_pad, src_pad, residual, w8, wc, pw_bf16, sc, bi)


def kernel(x, nbr_idx, dw_w1, pw_w1, bn1_g, bn1_b, bn1_m, bn1_v,
           dw_w2, pw_w2, bn2_g, bn2_b, bn2_m, bn2_v):
    n, c = x.shape
    assert c == 256 and n % B == 0
    nblk = n // B

    idxp = jnp.where(nbr_idx < 0, jnp.int32(n), nbr_idx.astype(jnp.int32))
    sel = jnp.concatenate([idxp[:4], idxp[5:]], axis=0)
    sel = jnp.pad(sel, ((0, 0), (0, B)))
    idx_flat = sel.reshape(8, nblk + 1, B).transpose(1, 0, 2)
    idx_flat = idx_flat.reshape(nblk + 1, 8 * B)

    xpad = jnp.concatenate([x, jnp.zeros((B, c), jnp.float32)], axis=0)

    s1 = (bn1_g * jax.lax.rsqrt(bn1_v + EPS)).reshape(1, c)
    o1 = (bn1_b - bn1_m * s1[0]).reshape(1, c)
    s2 = (bn2_g * jax.lax.rsqrt(bn2_v + EPS)).reshape(1, c)
    o2 = (bn2_b - bn2_m * s2[0]).reshape(1, c)

    w8_1 = jnp.concatenate([dw_w1[:4], dw_w1[5:]], axis=0)
    wc_1 = dw_w1[4:5]
    w8_2 = jnp.concatenate([dw_w2[:4], dw_w2[5:]], axis=0)
    wc_2 = dw_w2[4:5]

    pw1b = pw_w1.astype(jnp.bfloat16)
    pw2b = pw_w2.astype(jnp.bfloat16)

    h1pad = _run_half(
        idx_flat, xpad, xpad, w8_1, wc_1, pw1b, s1, o1,
        n_out_rows=n + B, nblk=nblk, grid=nblk + 1,
        relu_gather=True, relu_out=True, add_residual=False)

    out = _run_half(
        idx_flat, h1pad, x, w8_2, wc_2, pw2b, s2, o2,
        n_out_rows=n, nblk=nblk, grid=nblk,
        relu_gather=False, relu_out=False, add_residual=True)
    return out
